# Initial kernel scaffold; baseline (speedup 1.0000x reference)
import jax
import jax.numpy as jnp
from jax.experimental import pallas as pl


def kernel(position_ids, positional_encoding):
    raise NotImplementedError("write your pallas kernel here")


# speedup vs baseline: 2.1907x; 2.1907x over previous
import functools

import jax
import jax.numpy as jnp
from jax import lax
from jax.experimental import pallas as pl
from jax.experimental.pallas import tpu as pltpu
from jax.experimental.pallas import tpu_sc as plsc

_D = 1024
_NC, _NS = 2, 16
_NW = _NC * _NS
_CHUNK = 64


def _make_gather(n_rows: int):
  b_per_w = n_rows // _NW
  n_chunks = b_per_w // _CHUNK
  mesh = plsc.VectorSubcoreMesh(core_axis_name="c", subcore_axis_name="s")

  @functools.partial(
      pl.kernel,
      out_type=jax.ShapeDtypeStruct((n_rows, _D), jnp.float32),
      mesh=mesh,
      scratch_types=[
          pltpu.VMEM((b_per_w,), jnp.int32),
          pltpu.VMEM((_CHUNK, _D), jnp.float32),
          pltpu.SemaphoreType.DMA,
      ],
  )
  def gather_kernel(table_hbm, idx_hbm, out_hbm, idx_v, rows_v, sem):
    wid = lax.axis_index("s") * _NC + lax.axis_index("c")
    base = wid * b_per_w
    pltpu.sync_copy(idx_hbm.at[pl.ds(base, b_per_w)], idx_v)

    def body(c, carry):
      off = c * _CHUNK
      pltpu.async_copy(
          table_hbm.at[idx_v.at[pl.ds(off, _CHUNK)]], rows_v, sem
      ).wait()
      pltpu.sync_copy(rows_v, out_hbm.at[pl.ds(base + off, _CHUNK)])
      return carry

    lax.fori_loop(0, n_chunks, body, 0)

  return gather_kernel


def kernel(position_ids, positional_encoding):
  b, s = position_ids.shape
  flat_idx = position_ids.reshape(b * s).astype(jnp.int32)
  out = _make_gather(b * s)(positional_encoding, flat_idx)
  return out.reshape(b, s, positional_encoding.shape[1])


# speedup vs baseline: 2.3699x; 1.0818x over previous
import functools

import jax
import jax.numpy as jnp
from jax import lax
from jax.experimental import pallas as pl
from jax.experimental.pallas import tpu as pltpu
from jax.experimental.pallas import tpu_sc as plsc

_D = 1024
_NC, _NS = 2, 16
_NW = _NC * _NS
_CHUNK = 32


def _make_gather(n_rows: int):
  b_per_w = n_rows // _NW
  n_chunks = b_per_w // _CHUNK
  assert n_chunks % 2 == 0 and n_chunks >= 2
  mesh = plsc.VectorSubcoreMesh(core_axis_name="c", subcore_axis_name="s")

  @functools.partial(
      pl.kernel,
      out_type=jax.ShapeDtypeStruct((n_rows, _D), jnp.float32),
      mesh=mesh,
      scratch_types=[
          pltpu.VMEM((b_per_w,), jnp.int32),
          pltpu.VMEM((2, _CHUNK, _D), jnp.float32),
          pltpu.SemaphoreType.DMA,
          pltpu.SemaphoreType.DMA,
          pltpu.SemaphoreType.DMA,
          pltpu.SemaphoreType.DMA,
      ],
  )
  def gather_kernel(table_hbm, idx_hbm, out_hbm, idx_v, rows_v, gsem0, gsem1,
                    ssem0, ssem1):
    wid = lax.axis_index("s") * _NC + lax.axis_index("c")
    base = wid * b_per_w
    pltpu.sync_copy(idx_hbm.at[pl.ds(base, b_per_w)], idx_v)

    gsem = (gsem0, gsem1)
    ssem = (ssem0, ssem1)

    def gather_start(c, slot):
      pltpu.async_copy(
          table_hbm.at[idx_v.at[pl.ds(c * _CHUNK, _CHUNK)]],
          rows_v.at[slot], gsem[slot])

    def gather_wait(slot):
      pltpu.make_async_copy(
          table_hbm.at[idx_v.at[pl.ds(0, _CHUNK)]],
          rows_v.at[slot], gsem[slot]).wait()

    def scatter_start(c, slot):
      pltpu.async_copy(
          rows_v.at[slot], out_hbm.at[pl.ds(base + c * _CHUNK, _CHUNK)],
          ssem[slot])

    def scatter_wait(slot):
      pltpu.make_async_copy(
          rows_v.at[slot], out_hbm.at[pl.ds(base, _CHUNK)], ssem[slot]).wait()

    gather_start(0, 0)

    @pl.loop(0, n_chunks, step=2)
    def _body(c):
      @pl.when(c > 0)
      def _():
        scatter_wait(1)
      gather_start(c + 1, 1)
      gather_wait(0)
      scatter_start(c, 0)
      scatter_wait(0)
      @pl.when(c + 2 < n_chunks)
      def _():
        gather_start(c + 2, 0)
      gather_wait(1)
      scatter_start(c + 1, 1)

    scatter_wait(1)

  return gather_kernel


def kernel(position_ids, positional_encoding):
  b, s = position_ids.shape
  flat_idx = position_ids.reshape(b * s).astype(jnp.int32)
  out = _make_gather(b * s)(positional_encoding, flat_idx)
  return out.reshape(b, s, positional_encoding.shape[1])


# speedup vs baseline: 2.3877x; 1.0075x over previous
import functools

import jax
import jax.numpy as jnp
from jax import lax
from jax.experimental import pallas as pl
from jax.experimental.pallas import tpu as pltpu
from jax.experimental.pallas import tpu_sc as plsc

_D = 1024
_NC, _NS = 2, 16
_NW = _NC * _NS
_CHUNK = 16
_NSLOT = 4


def _make_gather(n_rows: int):
  b_per_w = n_rows // _NW
  n_chunks = b_per_w // _CHUNK
  assert n_chunks % _NSLOT == 0 and n_chunks >= _NSLOT
  mesh = plsc.VectorSubcoreMesh(core_axis_name="c", subcore_axis_name="s")

  @functools.partial(
      pl.kernel,
      out_type=jax.ShapeDtypeStruct((n_rows, _D), jnp.float32),
      mesh=mesh,
      scratch_types=[
          pltpu.VMEM((b_per_w,), jnp.int32),
          pltpu.VMEM((_NSLOT, _CHUNK, _D), jnp.float32),
          [pltpu.SemaphoreType.DMA] * _NSLOT,
          [pltpu.SemaphoreType.DMA] * _NSLOT,
      ],
  )
  def gather_kernel(table_hbm, idx_hbm, out_hbm, idx_v, rows_v, gsem, ssem):
    wid = lax.axis_index("s") * _NC + lax.axis_index("c")
    base = wid * b_per_w
    pltpu.sync_copy(idx_hbm.at[pl.ds(base, b_per_w)], idx_v)

    def gather_start(c, slot):
      pltpu.async_copy(
          table_hbm.at[idx_v.at[pl.ds(c * _CHUNK, _CHUNK)]],
          rows_v.at[slot], gsem[slot])

    def gather_wait(slot):
      pltpu.make_async_copy(
          table_hbm.at[idx_v.at[pl.ds(0, _CHUNK)]],
          rows_v.at[slot], gsem[slot]).wait()

    def scatter_start(c, slot):
      pltpu.async_copy(
          rows_v.at[slot], out_hbm.at[pl.ds(base + c * _CHUNK, _CHUNK)],
          ssem[slot])

    def scatter_wait(slot):
      pltpu.make_async_copy(
          rows_v.at[slot], out_hbm.at[pl.ds(base, _CHUNK)], ssem[slot]).wait()

    for j in range(_NSLOT - 1):
      gather_start(j, j)

    @pl.loop(0, n_chunks, step=_NSLOT)
    def _body(c):
      for b in range(_NSLOT):
        cb = c + b
        g = cb + _NSLOT - 1
        gslot = (_NSLOT - 1 + b) % _NSLOT

        def _launch():
          scatter_wait(gslot)
          gather_start(g, gslot)

        if b == 0:
          @pl.when(c > 0)
          def _():
            _launch()
          @pl.when(c == 0)
          def _():
            gather_start(g, gslot)
        else:
          @pl.when(g < n_chunks)
          def _():
            _launch()
        gather_wait(b)
        scatter_start(cb, b)

    for j in range(_NSLOT):
      scatter_wait(j)

  return gather_kernel


def kernel(position_ids, positional_encoding):
  b, s = position_ids.shape
  flat_idx = position_ids.reshape(b * s).astype(jnp.int32)
  out = _make_gather(b * s)(positional_encoding, flat_idx)
  return out.reshape(b, s, positional_encoding.shape[1])
